# Initial kernel scaffold; baseline (speedup 1.0000x reference)
import jax
import jax.numpy as jnp
from jax.experimental import pallas as pl


def kernel(ids_f0, ids_f1, lengths_f0, lengths_f1, table_f0, table_f1):
    raise NotImplementedError("write your pallas kernel here")


# speedup vs baseline: 1.6124x; 1.6124x over previous
import functools

import jax
import jax.numpy as jnp
from jax import lax
from jax.experimental import pallas as pl
from jax.experimental.pallas import tpu as pltpu
from jax.experimental.pallas import tpu_sc as plsc

BATCH = 4096
SEQ = 20
DIM = 64
NUM_CORES = 2
NUM_SUBCORES = 16
NW = NUM_CORES * NUM_SUBCORES
B_PER_W = BATCH // NW
CHUNK_B = 32
N_CHUNKS = B_PER_W // CHUNK_B
ROWS_PER_CHUNK = CHUNK_B * SEQ
IDX_MINOR = 128
N_GATHERS = ROWS_PER_CHUNK // IDX_MINOR
IDS_ROWS_PER_CHUNK = N_GATHERS


def _body(ids0_hbm, ids1_hbm, len0_hbm, len1_hbm, t0_hbm, t1_hbm, out_hbm,
          idx0_v, idx1_v, a_v, b_v, len0_v, len1_v, gsem, wsem):
  cid = lax.axis_index("c")
  sid = lax.axis_index("s")
  wid = sid * NUM_CORES + cid
  b0w = wid * B_PER_W

  pltpu.sync_copy(len0_hbm.at[pl.ds(b0w, B_PER_W)], len0_v)
  pltpu.sync_copy(len1_hbm.at[pl.ds(b0w, B_PER_W)], len1_v)

  zero = jnp.zeros((16,), jnp.float32)

  def chunk_body(c, carry):
    flat0 = (wid * B_PER_W + c * CHUNK_B) * SEQ
    pltpu.sync_copy(ids0_hbm.at[pl.ds(flat0, ROWS_PER_CHUNK)], idx0_v)
    pltpu.sync_copy(ids1_hbm.at[pl.ds(flat0, ROWS_PER_CHUNK)], idx1_v)

    handles = []
    for j in range(N_GATHERS):
      sl = pl.ds(j * IDX_MINOR, IDX_MINOR)
      handles.append(
          pltpu.async_copy(t0_hbm.at[idx0_v.at[sl]], a_v.at[sl], gsem))
      handles.append(
          pltpu.async_copy(t1_hbm.at[idx1_v.at[sl]], b_v.at[sl], gsem))
    for h in handles:
      h.wait()

    def zero_tail(ref, base_row, start):
      def zrow(s, _):
        r = base_row + s
        ref[r, pl.ds(0, 16)] = zero
        ref[r, pl.ds(16, 16)] = zero
        ref[r, pl.ds(32, 16)] = zero
        ref[r, pl.ds(48, 16)] = zero
        return 0
      lax.fori_loop(start, SEQ, zrow, 0)

    wh = []
    for half in range(CHUNK_B // 16):
      lv0 = len0_v[pl.ds(c * CHUNK_B + half * 16, 16)]
      lv1 = len1_v[pl.ds(c * CHUNK_B + half * 16, 16)]
      for lane in range(16):
        bi = half * 16 + lane
        zero_tail(a_v, bi * SEQ, lv0[lane])
        zero_tail(b_v, bi * SEQ, lv1[lane])

    for bi in range(CHUNK_B):
      gb = b0w + c * CHUNK_B + bi
      src = pl.ds(bi * SEQ, SEQ)
      wh.append(pltpu.async_copy(
          a_v.at[src], out_hbm.at[pl.ds(gb * 2 * SEQ, SEQ)], wsem))
      wh.append(pltpu.async_copy(
          b_v.at[src], out_hbm.at[pl.ds(gb * 2 * SEQ + SEQ, SEQ)], wsem))
    for h in wh:
      h.wait()
    return carry

  lax.fori_loop(0, N_CHUNKS, chunk_body, 0)


@jax.jit
def _run(ids_f0, ids_f1, lengths_f0, lengths_f1, table_f0, table_f1):
  mesh = plsc.VectorSubcoreMesh(core_axis_name="c", subcore_axis_name="s")
  ids0 = ids_f0.reshape(BATCH * SEQ)
  ids1 = ids_f1.reshape(BATCH * SEQ)
  out = pl.kernel(
      _body,
      out_type=jax.ShapeDtypeStruct((BATCH * 2 * SEQ, DIM), jnp.float32),
      mesh=mesh,
      compiler_params=pltpu.CompilerParams(use_tc_tiling_on_sc=False),
      scratch_types=[
          pltpu.VMEM((ROWS_PER_CHUNK,), jnp.int32),
          pltpu.VMEM((ROWS_PER_CHUNK,), jnp.int32),
          pltpu.VMEM((ROWS_PER_CHUNK, DIM), jnp.float32),
          pltpu.VMEM((ROWS_PER_CHUNK, DIM), jnp.float32),
          pltpu.VMEM((B_PER_W,), jnp.int32),
          pltpu.VMEM((B_PER_W,), jnp.int32),
          pltpu.SemaphoreType.DMA,
          pltpu.SemaphoreType.DMA,
      ],
  )(ids0, ids1, lengths_f0, lengths_f1, table_f0, table_f1)
  return out.reshape(BATCH, 2 * SEQ * DIM)


def kernel(ids_f0, ids_f1, lengths_f0, lengths_f1, table_f0, table_f1):
  return _run(ids_f0, ids_f1, lengths_f0, lengths_f1, table_f0, table_f1)
